# baseline (speedup 1.0000x reference)
import functools

import jax
import jax.numpy as jnp
from jax import lax
from jax.experimental import pallas as pl
from jax.experimental.pallas import tpu as pltpu
from jax.experimental.pallas import tpu_sc as plsc

N, T, NP, D_IN, D, TOPK = 8, 16, 196, 384, 4, 12
NF = N * T
FB = 8
NPAD = 208
NEG = -3.0e38

NC, NS, L = 2, 16, 16
NW = NC * NS
ROWS_PER_W = NF // NW
NCHUNK = NPAD // L


def _scores_body(x_ref, w_ref, out_ref):
    xb = x_ref[...].reshape(FB * NP, D_IN)
    kq = lax.dot_general(
        xb, w_ref[...], (((1,), (0,)), ((), ())),
        preferred_element_type=jnp.float32,
        precision=lax.Precision.DEFAULT)
    scale = 1.0 / jnp.sqrt(jnp.float32(D_IN))
    pad = jnp.full((NPAD - NP,), NEG, jnp.float32)
    for f in range(FB):
        k = kq[f * NP:(f + 1) * NP, :D]
        q = kq[f * NP:(f + 1) * NP, D:]
        h = lax.dot_general(
            k, q, (((1,), (1,)), ((), ())),
            preferred_element_type=jnp.float32,
            precision=lax.Precision.DEFAULT)
        s = h * scale
        m = jnp.max(s, axis=1, keepdims=True)
        e = jnp.exp(s - m)
        z = jnp.sum(e, axis=1, keepdims=True)
        a = jnp.sum(e / z, axis=0)
        out_ref[f, :] = jnp.concatenate([a, pad])


def _scores(xf, w):
    return pl.pallas_call(
        _scores_body,
        grid=(NF // FB,),
        in_specs=[
            pl.BlockSpec((FB, NP, D_IN), lambda i: (i, 0, 0)),
            pl.BlockSpec((D_IN, 2 * D), lambda i: (0, 0)),
        ],
        out_specs=pl.BlockSpec((FB, NPAD), lambda i: (i, 0)),
        out_shape=jax.ShapeDtypeStruct((NF, NPAD), jnp.float32),
    )(xf, w)


_GDN = lax.GatherDimensionNumbers(
    offset_dims=(), collapsed_slice_dims=(0,), start_index_map=(0,))


def _shuffle(v, idx):
    return lax.gather(v, idx[:, None], _GDN, slice_sizes=(1,),
                      mode=lax.GatherScatterMode.PROMISE_IN_BOUNDS)


def _allmax(v, perms):
    for p in perms:
        v = jnp.maximum(v, _shuffle(v, p))
    return v


def _topk_body(a_hbm, out_hbm, rows_v, idx_v):
    wid = lax.axis_index("s") * NC + lax.axis_index("c")
    base = wid * ROWS_PER_W
    pltpu.sync_copy(a_hbm.at[pl.ds(base, ROWS_PER_W)], rows_v)
    lane = lax.iota(jnp.int32, L)
    perms = [lane ^ d for d in (1, 2, 4, 8)]
    for r in range(ROWS_PER_W):
        vals = [rows_v[r, pl.ds(c * L, L)] for c in range(NCHUNK)]
        idxs = [lane + c * L for c in range(NCHUNK)]
        out_vec = jnp.zeros((L,), jnp.int32)

        def step(t, carry):
            vals_c = list(carry[:NCHUNK])
            out_c = carry[NCHUNK]
            m_val, m_idx = vals_c[0], idxs[0]
            for c in range(1, NCHUNK):
                take = vals_c[c] >= m_val
                m_val = jnp.where(take, vals_c[c], m_val)
                m_idx = jnp.where(take, idxs[c], m_idx)
            gm = _allmax(m_val, perms)
            gi = _allmax(jnp.where(m_val == gm, m_idx, -1), perms)
            out_c = jnp.where(lane == t, gi, out_c)
            for c in range(NCHUNK):
                vals_c[c] = jnp.where(idxs[c] == gi, NEG, vals_c[c])
            return tuple(vals_c) + (out_c,)

        res = lax.fori_loop(0, TOPK, step, tuple(vals) + (out_vec,))
        idx_v[r, :] = res[NCHUNK]
    pltpu.sync_copy(idx_v, out_hbm.at[pl.ds(base, ROWS_PER_W)])


def _topk(a):
    mesh = plsc.VectorSubcoreMesh(core_axis_name="c", subcore_axis_name="s")
    f = functools.partial(
        pl.kernel,
        out_type=jax.ShapeDtypeStruct((NF, L), jnp.int32),
        mesh=mesh,
        scratch_types=[
            pltpu.VMEM((ROWS_PER_W, NPAD), jnp.float32),
            pltpu.VMEM((ROWS_PER_W, L), jnp.int32),
        ],
    )(_topk_body)
    return f(a)


def kernel(x, wk, wq):
    xf = x.reshape(NF, NP, D_IN)
    w = jnp.concatenate([wk, wq], axis=1)
    a = _scores(xf, w)
    idx = _topk(a)
    return idx[:, :TOPK].reshape(N, T, TOPK, 1)


# speedup vs baseline: 1.0164x; 1.0164x over previous
import functools

import jax
import jax.numpy as jnp
from jax import lax
from jax.experimental import pallas as pl
from jax.experimental.pallas import tpu as pltpu
from jax.experimental.pallas import tpu_sc as plsc

N, T, NP, D_IN, D, TOPK = 8, 16, 196, 384, 4, 12
NF = N * T
FB = 8
NPAD = 208
NEG = -3.0e38

NC, NS, L = 2, 16, 16
NW = NC * NS
ROWS_PER_W = NF // NW
NCHUNK = NPAD // L


def _scores_body(x_ref, w_ref, out_ref):
    scale = 1.0 / jnp.sqrt(jnp.float32(D_IN))
    pad = jnp.full((NPAD - NP,), NEG, jnp.float32)
    w = w_ref[...]
    for t in range(T):
        xf = x_ref[0, :, t, :]
        kq = lax.dot_general(
            xf, w, (((1,), (0,)), ((), ())),
            preferred_element_type=jnp.float32,
            precision=lax.Precision.DEFAULT)
        k = kq[:, :D]
        q = kq[:, D:]
        h = lax.dot_general(
            k, q, (((1,), (1,)), ((), ())),
            preferred_element_type=jnp.float32,
            precision=lax.Precision.DEFAULT)
        s = h * scale
        m = jnp.max(s, axis=1, keepdims=True)
        e = jnp.exp(s - m)
        z = jnp.sum(e, axis=1, keepdims=True)
        a = jnp.sum(e / z, axis=0)
        out_ref[0, t, :] = jnp.concatenate([a, pad])


def _scores(xt, w):
    return pl.pallas_call(
        _scores_body,
        grid=(N,),
        in_specs=[
            pl.BlockSpec((1, NP, T, D_IN), lambda i: (i, 0, 0, 0)),
            pl.BlockSpec((D_IN, 2 * D), lambda i: (0, 0)),
        ],
        out_specs=pl.BlockSpec((1, T, NPAD), lambda i: (i, 0, 0)),
        out_shape=jax.ShapeDtypeStruct((N, T, NPAD), jnp.float32),
    )(xt, w)


_GDN = lax.GatherDimensionNumbers(
    offset_dims=(), collapsed_slice_dims=(0,), start_index_map=(0,))


def _shuffle(v, idx):
    return lax.gather(v, idx[:, None], _GDN, slice_sizes=(1,),
                      mode=lax.GatherScatterMode.PROMISE_IN_BOUNDS)


def _allmax(v, perms):
    for p in perms:
        v = jnp.maximum(v, _shuffle(v, p))
    return v


def _topk_body(a_hbm, out_hbm, rows_v, idx_v):
    wid = lax.axis_index("s") * NC + lax.axis_index("c")
    base = wid * ROWS_PER_W
    pltpu.sync_copy(a_hbm.at[pl.ds(base, ROWS_PER_W)], rows_v)
    lane = lax.iota(jnp.int32, L)
    perms = [lane ^ d for d in (1, 2, 4, 8)]
    for r in range(ROWS_PER_W):
        vals = [rows_v[r, pl.ds(c * L, L)] for c in range(NCHUNK)]
        idxs = [lane + c * L for c in range(NCHUNK)]
        out_vec = jnp.zeros((L,), jnp.int32)

        def step(t, carry):
            vals_c = list(carry[:NCHUNK])
            out_c = carry[NCHUNK]
            m_val, m_idx = vals_c[0], idxs[0]
            for c in range(1, NCHUNK):
                take = vals_c[c] >= m_val
                m_val = jnp.where(take, vals_c[c], m_val)
                m_idx = jnp.where(take, idxs[c], m_idx)
            gm = _allmax(m_val, perms)
            gi = _allmax(jnp.where(m_val == gm, m_idx, -1), perms)
            out_c = jnp.where(lane == t, gi, out_c)
            for c in range(NCHUNK):
                vals_c[c] = jnp.where(idxs[c] == gi, NEG, vals_c[c])
            return tuple(vals_c) + (out_c,)

        res = lax.fori_loop(0, TOPK, step, tuple(vals) + (out_vec,))
        idx_v[r, :] = res[NCHUNK]
    pltpu.sync_copy(idx_v, out_hbm.at[pl.ds(base, ROWS_PER_W)])


def _topk(a):
    mesh = plsc.VectorSubcoreMesh(core_axis_name="c", subcore_axis_name="s")
    f = functools.partial(
        pl.kernel,
        out_type=jax.ShapeDtypeStruct((NF, L), jnp.int32),
        mesh=mesh,
        scratch_types=[
            pltpu.VMEM((ROWS_PER_W, NPAD), jnp.float32),
            pltpu.VMEM((ROWS_PER_W, L), jnp.int32),
        ],
    )(_topk_body)
    return f(a)


def kernel(x, wk, wq):
    xt = jnp.transpose(x, (0, 2, 1, 3))
    w = jnp.concatenate([wk, wq], axis=1)
    a = _scores(xt, w).reshape(NF, NPAD)
    idx = _topk(a)
    return idx[:, :TOPK].reshape(N, T, TOPK, 1)


# speedup vs baseline: 1.8324x; 1.8028x over previous
import functools

import jax
import jax.numpy as jnp
from jax import lax
from jax.experimental import pallas as pl
from jax.experimental.pallas import tpu as pltpu
from jax.experimental.pallas import tpu_sc as plsc

N, T, NP, D_IN, D, TOPK = 8, 16, 196, 384, 4, 12
NF = N * T
FB = 8
NPAD = 208
NEG = -3.0e38

NC, NS, L = 2, 16, 16
NW = NC * NS
ROWS_PER_W = NF // NW
NCHUNK = NPAD // L


def _scores_body(x_ref, w_ref, out_ref):
    scale = 1.0 / jnp.sqrt(jnp.float32(D_IN))
    pad = jnp.full((NPAD - NP,), NEG, jnp.float32)
    xb = x_ref[0].reshape(NP * T, D_IN)
    kq_all = lax.dot_general(
        xb, w_ref[...], (((1,), (0,)), ((), ())),
        preferred_element_type=jnp.float32,
        precision=lax.Precision.DEFAULT).reshape(NP, T, 2 * D)
    for t in range(T):
        kq = kq_all[:, t, :]
        k = kq[:, :D]
        q = kq[:, D:]
        h = lax.dot_general(
            k, q, (((1,), (1,)), ((), ())),
            preferred_element_type=jnp.float32,
            precision=lax.Precision.DEFAULT)
        s = h * scale
        m = jnp.max(s, axis=1, keepdims=True)
        e = jnp.exp(s - m)
        z = jnp.sum(e, axis=1, keepdims=True)
        a = jnp.sum(e / z, axis=0)
        out_ref[0, t, :] = jnp.concatenate([a, pad])


def _scores(xt, w):
    return pl.pallas_call(
        _scores_body,
        grid=(N,),
        in_specs=[
            pl.BlockSpec((1, NP, T, D_IN), lambda i: (i, 0, 0, 0)),
            pl.BlockSpec((D_IN, 2 * D), lambda i: (0, 0)),
        ],
        out_specs=pl.BlockSpec((1, T, NPAD), lambda i: (i, 0, 0)),
        out_shape=jax.ShapeDtypeStruct((N, T, NPAD), jnp.float32),
    )(xt, w)


_GDN = lax.GatherDimensionNumbers(
    offset_dims=(), collapsed_slice_dims=(0,), start_index_map=(0,))


def _shuffle(v, idx):
    return lax.gather(v, idx[:, None], _GDN, slice_sizes=(1,),
                      mode=lax.GatherScatterMode.PROMISE_IN_BOUNDS)


def _allmax(v, perms):
    for p in perms:
        v = jnp.maximum(v, _shuffle(v, p))
    return v


def _topk_body(a_hbm, out_hbm, rows_v, idx_v):
    wid = lax.axis_index("s") * NC + lax.axis_index("c")
    base = wid * ROWS_PER_W
    pltpu.sync_copy(a_hbm.at[pl.ds(base, ROWS_PER_W)], rows_v)
    lane = lax.iota(jnp.int32, L)
    perms = [lane ^ d for d in (1, 2, 4, 8)]
    for r in range(ROWS_PER_W):
        vals = [rows_v[r, pl.ds(c * L, L)] for c in range(NCHUNK)]
        idxs = [lane + c * L for c in range(NCHUNK)]
        out_vec = jnp.zeros((L,), jnp.int32)

        def step(t, carry):
            vals_c = list(carry[:NCHUNK])
            out_c = carry[NCHUNK]
            m_val, m_idx = vals_c[0], idxs[0]
            for c in range(1, NCHUNK):
                take = vals_c[c] >= m_val
                m_val = jnp.where(take, vals_c[c], m_val)
                m_idx = jnp.where(take, idxs[c], m_idx)
            gm = _allmax(m_val, perms)
            gi = _allmax(jnp.where(m_val == gm, m_idx, -1), perms)
            out_c = jnp.where(lane == t, gi, out_c)
            for c in range(NCHUNK):
                vals_c[c] = jnp.where(idxs[c] == gi, NEG, vals_c[c])
            return tuple(vals_c) + (out_c,)

        res = lax.fori_loop(0, TOPK, step, tuple(vals) + (out_vec,))
        idx_v[r, :] = res[NCHUNK]
    pltpu.sync_copy(idx_v, out_hbm.at[pl.ds(base, ROWS_PER_W)])


def _topk(a):
    mesh = plsc.VectorSubcoreMesh(core_axis_name="c", subcore_axis_name="s")
    f = functools.partial(
        pl.kernel,
        out_type=jax.ShapeDtypeStruct((NF, L), jnp.int32),
        mesh=mesh,
        scratch_types=[
            pltpu.VMEM((ROWS_PER_W, NPAD), jnp.float32),
            pltpu.VMEM((ROWS_PER_W, L), jnp.int32),
        ],
    )(_topk_body)
    return f(a)


def kernel(x, wk, wq):
    xt = jnp.transpose(x, (0, 2, 1, 3))
    w = jnp.concatenate([wk, wq], axis=1)
    a = _scores(xt, w).reshape(NF, NPAD)
    idx = _topk(a)
    return idx[:, :TOPK].reshape(N, T, TOPK, 1)
